# baseline (speedup 1.0000x reference)
import jax
import jax.numpy as jnp
from jax import lax
from jax.experimental import pallas as pl
from jax.experimental.pallas import tpu as pltpu
from jax.experimental.pallas import tpu_sc as plsc

_R = 3
_BW = 0.1
_C, _H, _W = 96, 224, 224
_HW = _H * _W
_PAD = 16
_LANES = 16
_NWORK = 32
_NSC = 64
_CPW = _NSC // _NWORK
_NVEC = _W // _LANES


def _body(x_hbm, out_hbm, xbuf, obuf, sem):
    del sem
    wid = lax.axis_index("s") * 2 + lax.axis_index("c")
    zero16 = jnp.zeros((_LANES,), jnp.float32)
    lane = lax.iota(jnp.int32, _LANES)

    for k in range(_CPW):
        ch = wid * _CPW + k
        pltpu.sync_copy(x_hbm.at[ch], xbuf.at[pl.ds(_PAD, _HW)])

        for jv in range(_NVEC):
            obuf[pl.ds(jv * _LANES, _LANES)] = zero16
            obuf[pl.ds((_H - 1) * _W + jv * _LANES, _LANES)] = zero16

        for jv in range(_NVEC):
            col0 = jv * _LANES

            @plsc.parallel_loop(1, _H - 1, step=1, unroll=3)
            def row_body(i, col0=col0, jv=jv):
                base = i * _W + col0 + _PAD
                c = xbuf[pl.ds(base, _LANES)]
                acc = jnp.zeros((_LANES,), jnp.float32)
                for di in (-1, 0, 1):
                    for dj in (-1, 0, 1):
                        if di == 0 and dj == 0:
                            continue
                        v = xbuf[pl.ds(base + di * _W + dj, _LANES)]
                        d = (c - v) if (di, dj) < (0, 0) else (v - c)
                        acc = acc + jnp.maximum(0.0, _BW - jnp.abs(d))
                acc = acc * jnp.float32(1.0 / (_BW * _R * _R)) + jnp.float32(
                    1.0 / (_R * _R))
                if jv == 0:
                    acc = jnp.where(lane >= 1, acc, 0.0)
                if jv == _NVEC - 1:
                    acc = jnp.where(lane <= _LANES - 2, acc, 0.0)
                obuf[pl.ds(i * _W + col0, _LANES)] = acc

        pltpu.sync_copy(obuf, out_hbm.at[ch])


def _hist_sc(x2d):
    mesh = plsc.VectorSubcoreMesh(core_axis_name="c", subcore_axis_name="s")
    f = pl.kernel(
        _body,
        out_type=jax.ShapeDtypeStruct((_NSC, _HW), jnp.float32),
        mesh=mesh,
        scratch_types=[
            pltpu.VMEM((_PAD + _HW + _PAD,), jnp.float32),
            pltpu.VMEM((_HW,), jnp.float32),
            pltpu.SemaphoreType.DMA,
        ],
        compiler_params=pltpu.CompilerParams(use_tc_tiling_on_sc=False),
    )
    return f(x2d)


_BC = 8


def _tc_body(x_ref, o_ref):
    x = x_ref[...]
    rows = {di: jnp.roll(x, -di, axis=1) if di else x for di in (-1, 0, 1)}
    acc = jnp.zeros_like(x)
    for di in (-1, 0, 1):
        for dj in (-1, 0, 1):
            if di == 0 and dj == 0:
                continue
            v = jnp.roll(rows[di], -dj, axis=2) if dj else rows[di]
            acc = acc + jnp.maximum(0.0, _BW - jnp.abs(v - x))
    acc = acc * jnp.float32(1.0 / (_BW * _R * _R)) + jnp.float32(1.0 / (_R * _R))
    row = lax.broadcasted_iota(jnp.int32, x.shape, 1)
    col = lax.broadcasted_iota(jnp.int32, x.shape, 2)
    interior = ((row >= 1) & (row <= _H - 2)) & ((col >= 1) & (col <= _W - 2))
    o_ref[...] = jnp.where(interior, acc, 0.0)


def _hist_tc(x):
    ct = x.shape[0]
    return pl.pallas_call(
        _tc_body,
        out_shape=jax.ShapeDtypeStruct((ct, _H, _W), jnp.float32),
        grid=(ct // _BC,),
        in_specs=[pl.BlockSpec((_BC, _H, _W), lambda i: (i, 0, 0))],
        out_specs=pl.BlockSpec((_BC, _H, _W), lambda i: (i, 0, 0)),
    )(x)


@jax.jit
def _hist(x3):
    sc_out = _hist_sc(x3[:_NSC].reshape(_NSC, _HW)).reshape(_NSC, _H, _W)
    tc_out = _hist_tc(x3[_NSC:])
    return jnp.concatenate([sc_out, tc_out], axis=0)


def kernel(input):
    n, sf, c, h, w = input.shape
    out = _hist(input.reshape(c, h, w))
    return out.reshape(n, sf, c, h, w)


# speedup vs baseline: 1.2722x; 1.2722x over previous
import jax
import jax.numpy as jnp
from jax import lax
from jax.experimental import pallas as pl
from jax.experimental.pallas import tpu as pltpu
from jax.experimental.pallas import tpu_sc as plsc

_R = 3
_BW = 0.1
_C, _H, _W = 96, 224, 224
_HW = _H * _W
_PAD = 16
_LANES = 16
_NWORK = 32
_NSC = 32
_CPW = _NSC // _NWORK
_NVEC = _W // _LANES


def _body(x_hbm, out_hbm, xbuf, obuf, sem):
    del sem
    wid = lax.axis_index("s") * 2 + lax.axis_index("c")
    zero16 = jnp.zeros((_LANES,), jnp.float32)
    lane = lax.iota(jnp.int32, _LANES)

    for k in range(_CPW):
        ch = wid * _CPW + k
        pltpu.sync_copy(x_hbm.at[ch], xbuf.at[pl.ds(_PAD, _HW)])

        for jv in range(_NVEC):
            obuf[pl.ds(jv * _LANES, _LANES)] = zero16
            obuf[pl.ds((_H - 1) * _W + jv * _LANES, _LANES)] = zero16

        for jv in range(_NVEC):
            col0 = jv * _LANES

            @plsc.parallel_loop(1, _H - 1, step=1, unroll=3)
            def row_body(i, col0=col0, jv=jv):
                base = i * _W + col0 + _PAD
                c = xbuf[pl.ds(base, _LANES)]
                acc = jnp.zeros((_LANES,), jnp.float32)
                for di in (-1, 0, 1):
                    for dj in (-1, 0, 1):
                        if di == 0 and dj == 0:
                            continue
                        v = xbuf[pl.ds(base + di * _W + dj, _LANES)]
                        d = (c - v) if (di, dj) < (0, 0) else (v - c)
                        acc = acc + jnp.maximum(0.0, _BW - jnp.abs(d))
                acc = acc * jnp.float32(1.0 / (_BW * _R * _R)) + jnp.float32(
                    1.0 / (_R * _R))
                if jv == 0:
                    acc = jnp.where(lane >= 1, acc, 0.0)
                if jv == _NVEC - 1:
                    acc = jnp.where(lane <= _LANES - 2, acc, 0.0)
                obuf[pl.ds(i * _W + col0, _LANES)] = acc

        pltpu.sync_copy(obuf, out_hbm.at[ch])


def _hist_sc(x2d):
    mesh = plsc.VectorSubcoreMesh(core_axis_name="c", subcore_axis_name="s")
    f = pl.kernel(
        _body,
        out_type=jax.ShapeDtypeStruct((_NSC, _HW), jnp.float32),
        mesh=mesh,
        scratch_types=[
            pltpu.VMEM((_PAD + _HW + _PAD,), jnp.float32),
            pltpu.VMEM((_HW,), jnp.float32),
            pltpu.SemaphoreType.DMA,
        ],
        compiler_params=pltpu.CompilerParams(use_tc_tiling_on_sc=False),
    )
    return f(x2d)


_BC = 8


def _tc_body(x_ref, o_ref):
    x = x_ref[...]
    xp = jnp.roll(x, -1, axis=1)

    def kval(v):
        return jnp.maximum(0.0, _BW - jnp.abs(v - x))

    ke = kval(jnp.roll(x, -1, axis=2))
    ks = kval(xp)
    kse = kval(jnp.roll(xp, -1, axis=2))
    ksw = kval(jnp.roll(xp, 1, axis=2))
    acc = (ke + ks) + (kse + ksw)
    acc = acc + jnp.roll(ke, 1, axis=2)
    acc = acc + jnp.roll(ks, 1, axis=1)
    acc = acc + jnp.roll(jnp.roll(kse, 1, axis=1), 1, axis=2)
    acc = acc + jnp.roll(jnp.roll(ksw, 1, axis=1), -1, axis=2)
    acc = acc * jnp.float32(1.0 / (_BW * _R * _R)) + jnp.float32(1.0 / (_R * _R))
    row = lax.broadcasted_iota(jnp.int32, x.shape, 1)
    col = lax.broadcasted_iota(jnp.int32, x.shape, 2)
    interior = ((row >= 1) & (row <= _H - 2)) & ((col >= 1) & (col <= _W - 2))
    o_ref[...] = jnp.where(interior, acc, 0.0)


def _hist_tc(x):
    ct = x.shape[0]
    return pl.pallas_call(
        _tc_body,
        out_shape=jax.ShapeDtypeStruct((ct, _H, _W), jnp.float32),
        grid=(ct // _BC,),
        in_specs=[pl.BlockSpec((_BC, _H, _W), lambda i: (i, 0, 0))],
        out_specs=pl.BlockSpec((_BC, _H, _W), lambda i: (i, 0, 0)),
    )(x)


@jax.jit
def _hist(x3):
    sc_out = _hist_sc(x3[:_NSC].reshape(_NSC, _HW)).reshape(_NSC, _H, _W)
    tc_out = _hist_tc(x3[_NSC:])
    return jnp.concatenate([sc_out, tc_out], axis=0)


def kernel(input):
    n, sf, c, h, w = input.shape
    out = _hist(input.reshape(c, h, w))
    return out.reshape(n, sf, c, h, w)


# speedup vs baseline: 1.4129x; 1.1106x over previous
import jax
import jax.numpy as jnp
from jax import lax
from jax.experimental import pallas as pl
from jax.experimental.pallas import tpu as pltpu
from jax.experimental.pallas import tpu_sc as plsc

_R = 3
_BW = 0.1
_C, _H, _W = 96, 224, 224
_HW = _H * _W
_PAD = 16
_LANES = 16
_NWORK = 32
_NSC = 32
_CPW = _NSC // _NWORK
_NVEC = _W // _LANES


def _body(x_hbm, out_hbm, xbuf, obuf, sem):
    del sem
    wid = lax.axis_index("s") * 2 + lax.axis_index("c")
    zero16 = jnp.zeros((_LANES,), jnp.float32)
    lane = lax.iota(jnp.int32, _LANES)

    for k in range(_CPW):
        ch = wid * _CPW + k
        pltpu.sync_copy(x_hbm.at[ch], xbuf.at[pl.ds(_PAD, _HW)])

        for jv in range(_NVEC):
            obuf[pl.ds(jv * _LANES, _LANES)] = zero16
            obuf[pl.ds((_H - 1) * _W + jv * _LANES, _LANES)] = zero16

        for jv in range(_NVEC):
            col0 = jv * _LANES

            @plsc.parallel_loop(1, _H - 1, step=1, unroll=3)
            def row_body(i, col0=col0, jv=jv):
                base = i * _W + col0 + _PAD
                c = xbuf[pl.ds(base, _LANES)]
                acc = jnp.zeros((_LANES,), jnp.float32)
                for di in (-1, 0, 1):
                    for dj in (-1, 0, 1):
                        if di == 0 and dj == 0:
                            continue
                        v = xbuf[pl.ds(base + di * _W + dj, _LANES)]
                        d = (c - v) if (di, dj) < (0, 0) else (v - c)
                        acc = acc + jnp.maximum(0.0, _BW - jnp.abs(d))
                acc = acc * jnp.float32(1.0 / (_BW * _R * _R)) + jnp.float32(
                    1.0 / (_R * _R))
                if jv == 0:
                    acc = jnp.where(lane >= 1, acc, 0.0)
                if jv == _NVEC - 1:
                    acc = jnp.where(lane <= _LANES - 2, acc, 0.0)
                obuf[pl.ds(i * _W + col0, _LANES)] = acc

        pltpu.sync_copy(obuf, out_hbm.at[ch])


def _hist_sc(x2d):
    mesh = plsc.VectorSubcoreMesh(core_axis_name="c", subcore_axis_name="s")
    f = pl.kernel(
        _body,
        out_type=jax.ShapeDtypeStruct((_NSC, _HW), jnp.float32),
        mesh=mesh,
        scratch_types=[
            pltpu.VMEM((_PAD + _HW + _PAD,), jnp.float32),
            pltpu.VMEM((_HW,), jnp.float32),
            pltpu.SemaphoreType.DMA,
        ],
        compiler_params=pltpu.CompilerParams(use_tc_tiling_on_sc=False),
    )
    return f(x2d)


_BC = 8


def _tc_body(x_ref, o_ref):
    x = x_ref[...]
    xl = jnp.roll(x, 1, axis=2)
    xr = jnp.roll(x, -1, axis=2)

    def kval(v):
        return jnp.maximum(0.0, _BW - jnp.abs(v - x))

    acc = kval(xl) + kval(xr)
    for base in (x, xl, xr):
        acc = acc + kval(jnp.roll(base, 1, axis=1))
        acc = acc + kval(jnp.roll(base, -1, axis=1))
    acc = acc * jnp.float32(1.0 / (_BW * _R * _R)) + jnp.float32(1.0 / (_R * _R))
    row = lax.broadcasted_iota(jnp.int32, x.shape, 1)
    col = lax.broadcasted_iota(jnp.int32, x.shape, 2)
    interior = ((row >= 1) & (row <= _H - 2)) & ((col >= 1) & (col <= _W - 2))
    o_ref[...] = jnp.where(interior, acc, 0.0)


def _hist_tc(x):
    ct = x.shape[0]
    return pl.pallas_call(
        _tc_body,
        out_shape=jax.ShapeDtypeStruct((ct, _H, _W), jnp.float32),
        grid=(ct // _BC,),
        in_specs=[pl.BlockSpec((_BC, _H, _W), lambda i: (i, 0, 0))],
        out_specs=pl.BlockSpec((_BC, _H, _W), lambda i: (i, 0, 0)),
    )(x)


@jax.jit
def _hist(x3):
    sc_out = _hist_sc(x3[:_NSC].reshape(_NSC, _HW)).reshape(_NSC, _H, _W)
    tc_out = _hist_tc(x3[_NSC:])
    return jnp.concatenate([sc_out, tc_out], axis=0)


def kernel(input):
    n, sf, c, h, w = input.shape
    out = _hist(input.reshape(c, h, w))
    return out.reshape(n, sf, c, h, w)
